# baseline (speedup 1.0000x reference)
import functools

import jax
import jax.numpy as jnp
from jax import lax
from jax.experimental import pallas as pl
from jax.experimental.pallas import tpu as pltpu
from jax.experimental.pallas import tpu_sc as plsc

NUM_PATCHES = 1024
D_MODEL = 768
BATCH = 64

SC_BATCH = 16
TC_BATCH = BATCH - SC_BATCH

NC = 2
NS = 16
NW = NC * NS
P_PER_W = NUM_PATCHES // NW
LANES = 16
SLICES_PER_ROW = D_MODEL // LANES


def _tc_body(x_ref, pos_ref, o_ref):
    o_ref[...] = x_ref[...] + pos_ref[...]


def _tc_add(x_tc, pos_table):
    bb = 4
    return pl.pallas_call(
        _tc_body,
        grid=(TC_BATCH // bb,),
        in_specs=[
            pl.BlockSpec((bb, NUM_PATCHES, D_MODEL), lambda b: (b, 0, 0)),
            pl.BlockSpec((NUM_PATCHES, D_MODEL), lambda b: (0, 0)),
        ],
        out_specs=pl.BlockSpec((bb, NUM_PATCHES, D_MODEL), lambda b: (b, 0, 0)),
        out_shape=jax.ShapeDtypeStruct((TC_BATCH, NUM_PATCHES, D_MODEL), jnp.float32),
    )(x_tc, pos_table)


_SC_MESH = plsc.VectorSubcoreMesh(core_axis_name="c", subcore_axis_name="s")


@functools.partial(
    pl.kernel,
    out_type=jax.ShapeDtypeStruct((SC_BATCH, NUM_PATCHES, D_MODEL), jnp.float32),
    mesh=_SC_MESH,
    scratch_types=[
        pltpu.VMEM((P_PER_W, D_MODEL), jnp.float32),
        pltpu.VMEM((P_PER_W, D_MODEL), jnp.float32),
    ],
)
def _sc_add(x_hbm, pos_hbm, out_hbm, pos_v, buf_v):
    wid = lax.axis_index("s") * NC + lax.axis_index("c")
    p0 = wid * P_PER_W
    pltpu.sync_copy(pos_hbm.at[pl.ds(p0, P_PER_W)], pos_v)

    def per_batch(b, _):
        pltpu.sync_copy(x_hbm.at[b, pl.ds(p0, P_PER_W)], buf_v)

        def per_row(p, _):
            def per_slice(j, _):
                sl = pl.ds(j * LANES, LANES)
                buf_v[p, sl] = buf_v[p, sl] + pos_v[p, sl]
                return ()
            return lax.fori_loop(0, SLICES_PER_ROW, per_slice, (), unroll=8)

        lax.fori_loop(0, P_PER_W, per_row, ())
        pltpu.sync_copy(buf_v, out_hbm.at[b, pl.ds(p0, P_PER_W)])
        return ()

    lax.fori_loop(0, SC_BATCH, per_batch, ())


def kernel(x, pos_table):
    out_tc = _tc_add(x[:TC_BATCH], pos_table)
    out_sc = _sc_add(x[TC_BATCH:], pos_table)
    return jnp.concatenate([out_tc, out_sc], axis=0)


# speedup vs baseline: 1.0024x; 1.0024x over previous
import functools

import jax
import jax.numpy as jnp
from jax import lax
from jax.experimental import pallas as pl
from jax.experimental.pallas import tpu as pltpu
from jax.experimental.pallas import tpu_sc as plsc

NUM_PATCHES = 1024
D_MODEL = 768
BATCH = 64

SC_BATCH = 16
TC_BATCH = BATCH - SC_BATCH

NC = 2
NS = 16
NW = NC * NS
P_PER_W = NUM_PATCHES // NW
LANES = 16
SLICES_PER_ROW = D_MODEL // LANES


def _tc_body(x_ref, pos_ref, o_ref):
    o_ref[...] = x_ref[...] + pos_ref[...]


def _tc_add(x_tc, pos_table):
    bb = 4
    return pl.pallas_call(
        _tc_body,
        grid=(TC_BATCH // bb,),
        in_specs=[
            pl.BlockSpec((bb, NUM_PATCHES, D_MODEL), lambda b: (b, 0, 0)),
            pl.BlockSpec((NUM_PATCHES, D_MODEL), lambda b: (0, 0)),
        ],
        out_specs=pl.BlockSpec((bb, NUM_PATCHES, D_MODEL), lambda b: (b, 0, 0)),
        out_shape=jax.ShapeDtypeStruct((TC_BATCH, NUM_PATCHES, D_MODEL), jnp.float32),
    )(x_tc, pos_table)


_SC_MESH = plsc.VectorSubcoreMesh(core_axis_name="c", subcore_axis_name="s")


@functools.partial(
    pl.kernel,
    out_type=jax.ShapeDtypeStruct((SC_BATCH, NUM_PATCHES, D_MODEL), jnp.float32),
    mesh=_SC_MESH,
    scratch_types=[
        pltpu.VMEM((P_PER_W, D_MODEL), jnp.float32),
        pltpu.VMEM((P_PER_W, D_MODEL), jnp.float32),
        pltpu.VMEM((P_PER_W, D_MODEL), jnp.float32),
        pltpu.SemaphoreType.DMA,
        pltpu.SemaphoreType.DMA,
        pltpu.SemaphoreType.DMA,
        pltpu.SemaphoreType.DMA,
    ],
)
def _sc_add(x_hbm, pos_hbm, out_hbm, pos_v, buf0, buf1, si0, si1, so0, so1):
    wid = lax.axis_index("s") * NC + lax.axis_index("c")
    p0 = wid * P_PER_W
    pltpu.sync_copy(pos_hbm.at[pl.ds(p0, P_PER_W)], pos_v)

    bufs = (buf0, buf1)
    sin = (si0, si1)
    sout = (so0, so1)
    in_h = [None, None]
    out_h = [None, None]

    def add_pos(buf):
        def per_row(p, _):
            def per_slice(j, _):
                sl = pl.ds(j * LANES, LANES)
                buf[p, sl] = buf[p, sl] + pos_v[p, sl]
                return ()
            return lax.fori_loop(0, SLICES_PER_ROW, per_slice, (), unroll=8)
        lax.fori_loop(0, P_PER_W, per_row, ())

    in_h[0] = pltpu.async_copy(x_hbm.at[0, pl.ds(p0, P_PER_W)], buf0, si0)
    for b in range(SC_BATCH):
        cur = b & 1
        nxt = cur ^ 1
        if b + 1 < SC_BATCH:
            if out_h[nxt] is not None:
                out_h[nxt].wait()
            in_h[nxt] = pltpu.async_copy(
                x_hbm.at[b + 1, pl.ds(p0, P_PER_W)], bufs[nxt], sin[nxt])
        in_h[cur].wait()
        add_pos(bufs[cur])
        out_h[cur] = pltpu.async_copy(
            bufs[cur], out_hbm.at[b, pl.ds(p0, P_PER_W)], sout[cur])
    out_h[0].wait()
    out_h[1].wait()


def kernel(x, pos_table):
    out_tc = _tc_add(x[:TC_BATCH], pos_table)
    out_sc = _sc_add(x[TC_BATCH:], pos_table)
    return jnp.concatenate([out_tc, out_sc], axis=0)
